# Initial kernel scaffold; baseline (speedup 1.0000x reference)
import jax
import jax.numpy as jnp
from jax.experimental import pallas as pl


def kernel(grad_loss, log_softmax, target, grad_zloss, lse_for_zloss):
    raise NotImplementedError("write your pallas kernel here")


# speedup vs baseline: 1.0107x; 1.0107x over previous
import functools

import jax
import jax.numpy as jnp
from jax.experimental import pallas as pl
from jax.experimental.pallas import tpu as pltpu

LABEL_SMOOTHING = 0.1
BB = 256
VB = 2048


def _ce_bwd_block(tgt_ref, gl_ref, ls_ref, out_ref, *, num_classes):
    j = pl.program_id(1)
    gl = gl_ref[...]
    a = gl * (1.0 - LABEL_SMOOTHING)
    c = gl * (LABEL_SMOOTHING / num_classes)
    ids = jax.lax.broadcasted_iota(jnp.int32, out_ref.shape, 1) + j * VB
    onehot = (ids == tgt_ref[...]).astype(jnp.float32)
    out_ref[...] = a * (jnp.exp(ls_ref[...]) - onehot) + c


def kernel(grad_loss, log_softmax, target, grad_zloss, lse_for_zloss):
    batch, num_classes = log_softmax.shape
    gl2 = grad_loss.astype(jnp.float32).reshape(batch, 1)
    tgt2 = target.astype(jnp.int32).reshape(batch, 1)
    grid = (batch // BB, pl.cdiv(num_classes, VB))
    out = pl.pallas_call(
        functools.partial(_ce_bwd_block, num_classes=num_classes),
        grid=grid,
        in_specs=[
            pl.BlockSpec((BB, 1), lambda i, j: (i, 0)),
            pl.BlockSpec((BB, 1), lambda i, j: (i, 0)),
            pl.BlockSpec((BB, VB), lambda i, j: (i, j)),
        ],
        out_specs=pl.BlockSpec((BB, VB), lambda i, j: (i, j)),
        out_shape=jax.ShapeDtypeStruct((batch, num_classes), jnp.float32),
        compiler_params=pltpu.CompilerParams(
            dimension_semantics=("parallel", "arbitrary"),
        ),
    )(tgt2, gl2, log_softmax.astype(jnp.float32))
    return out.astype(log_softmax.dtype)


# speedup vs baseline: 1.0615x; 1.0502x over previous
import functools

import jax
import jax.numpy as jnp
from jax.experimental import pallas as pl
from jax.experimental.pallas import tpu as pltpu

LABEL_SMOOTHING = 0.1
BB = 16
VB = 100000


def _ce_bwd_block(tgt_ref, gl_ref, ls_ref, out_ref, *, num_classes):
    j = pl.program_id(1)
    gl = gl_ref[...]
    a = gl * (1.0 - LABEL_SMOOTHING)
    c = gl * (LABEL_SMOOTHING / num_classes)
    ids = jax.lax.broadcasted_iota(jnp.int32, out_ref.shape, 1) + j * VB
    onehot = (ids == tgt_ref[...]).astype(jnp.float32)
    out_ref[...] = a * (jnp.exp(ls_ref[...]) - onehot) + c


def kernel(grad_loss, log_softmax, target, grad_zloss, lse_for_zloss):
    batch, num_classes = log_softmax.shape
    gl2 = grad_loss.astype(jnp.float32).reshape(batch, 1)
    tgt2 = target.astype(jnp.int32).reshape(batch, 1)
    grid = (batch // BB, pl.cdiv(num_classes, VB))
    out = pl.pallas_call(
        functools.partial(_ce_bwd_block, num_classes=num_classes),
        grid=grid,
        in_specs=[
            pl.BlockSpec((BB, 1), lambda i, j: (i, 0)),
            pl.BlockSpec((BB, 1), lambda i, j: (i, 0)),
            pl.BlockSpec((BB, VB), lambda i, j: (i, j)),
        ],
        out_specs=pl.BlockSpec((BB, VB), lambda i, j: (i, j)),
        out_shape=jax.ShapeDtypeStruct((batch, num_classes), jnp.float32),
        compiler_params=pltpu.CompilerParams(
            dimension_semantics=("parallel", "arbitrary"),
        ),
    )(tgt2, gl2, log_softmax.astype(jnp.float32))
    return out.astype(log_softmax.dtype)
